# baseline (speedup 1.0000x reference)
import functools

import jax
import jax.numpy as jnp
from jax import lax
from jax.experimental import pallas as pl
from jax.experimental.pallas import tpu as pltpu
from jax.experimental.pallas import tpu_sc as plsc

B = 16384
E = 128
NEIGH = 25
NL = 3
NW = 32
EPW = B // NW
CH = 4
ROWS = CH * NEIGH
NCH = EPW // CH
TB = 2048
NT = B // TB
WPT = TB // EPW


def _sc_gather_mean(batch2d, neigh4d, embedding, epw, seg):
    mesh = plsc.VectorSubcoreMesh(core_axis_name="c", subcore_axis_name="s")
    n_self = epw // 128
    nch = epw // CH
    nt = (NW * epw) // TB
    wpt = TB // epw

    @functools.partial(
        pl.kernel,
        out_type=(
            jax.ShapeDtypeStruct((nt, TB, E), jnp.float32),
            jax.ShapeDtypeStruct((nt, NL, TB, E), jnp.float32),
        ),
        mesh=mesh,
        scratch_types=[
            pltpu.VMEM((n_self, 128), jnp.int32),
            pltpu.VMEM((nch // 2, ROWS), jnp.int32),
        ] + [pltpu.VMEM((ROWS, E), jnp.float32) for _ in range(4)] + [
            pltpu.VMEM((epw, E), jnp.float32),
        ] + [pltpu.SemaphoreType.DMA for _ in range(4)],
    )
    def k(batch_hbm, neigh_hbm, emb_hbm, out_self, out_sums,
          sidx_v, idx_v, r0b, r1b, r2b, r3b, out_v,
          s0, s1, s2, s3):
        wid = lax.axis_index("s") * 2 + lax.axis_index("c")
        tile = wid // wpt
        trow = (wid % wpt) * epw
        bufs = (r0b, r1b, r2b, r3b)
        sems = (s0, s1, s2, s3)

        pltpu.sync_copy(batch_hbm.at[seg, wid], sidx_v)
        for c in range(n_self):
            pltpu.async_copy(emb_hbm.at[sidx_v.at[c]],
                             out_v.at[pl.ds(c * 128, 128), :], s0)
        for c in range(n_self):
            pltpu.make_async_copy(emb_hbm.at[sidx_v.at[c]],
                                  out_v.at[pl.ds(c * 128, 128), :], s0).wait()
        pltpu.sync_copy(out_v, out_self.at[tile, pl.ds(trow, epw), :])

        def accumulate(buf, g):
            for e in range(CH):
                r0 = e * NEIGH
                accs = tuple(buf[r0, pl.ds(r * 16, 16)] for r in range(8))

                def jbody(j, a):
                    return tuple(a[r] + buf[j, pl.ds(r * 16, 16)]
                                 for r in range(8))

                accs = lax.fori_loop(r0 + 1, r0 + NEIGH, jbody, accs,
                                     unroll=4)
                orow = g * CH + e
                for r in range(8):
                    out_v[orow, pl.ds(r * 16, 16)] = accs[r]

        NB = 4
        HC = nch // 2
        for layer in range(NL):
            for half in range(2):
                pltpu.sync_copy(neigh_hbm.at[layer, seg, wid,
                                             pl.ds(half * HC, HC)], idx_v)
                for b in range(NB):
                    pltpu.async_copy(emb_hbm.at[idx_v.at[b]], bufs[b], sems[b])

                def pbody(p, _):
                    for b in range(NB):
                        c = NB * p + b
                        pltpu.make_async_copy(emb_hbm.at[idx_v.at[c]],
                                              bufs[b], sems[b]).wait()
                        accumulate(bufs[b], half * HC + c)

                        @pl.when(c + NB < HC)
                        def _():
                            pltpu.async_copy(emb_hbm.at[idx_v.at[c + NB]],
                                             bufs[b], sems[b])
                    return 0

                lax.fori_loop(0, HC // NB, pbody, 0)
            pltpu.sync_copy(out_v,
                            out_sums.at[tile, layer, pl.ds(trow, epw), :])

    return k(batch2d, neigh4d, embedding)


def _tc_mlp(self_vec, sums, ws0, wn0, ws1, wn1, ws2, wn2):
    def body(s_ref, m_ref, ws0r, wn0r, ws1r, wn1r, ws2r, wn2r, o_ref):
        def dot(a, w):
            return jnp.dot(a, w, preferred_element_type=jnp.float32)

        h = jnp.maximum(
            dot(s_ref[0], ws0r[0])
            + dot(m_ref[0, 0], wn0r[0]), 0.0)
        h = jnp.maximum(
            dot(h, ws1r[0])
            + dot(m_ref[0, 1], wn1r[0]), 0.0)
        o_ref[0] = jnp.maximum(
            dot(h, ws2r[0])
            + dot(m_ref[0, 2], wn2r[0]), 0.0)

    def wspec(w):
        return pl.BlockSpec((1,) + w.shape, lambda i: (0, 0, 0))

    ws = [w[None] for w in (ws0, wn0, ws1, wn1, ws2, wn2)]
    nt = self_vec.shape[0]
    return pl.pallas_call(
        body,
        grid=(nt,),
        in_specs=[
            pl.BlockSpec((1, TB, E), lambda i: (i, 0, 0)),
            pl.BlockSpec((1, NL, TB, E), lambda i: (i, 0, 0, 0)),
        ] + [wspec(w) for w in (ws0, wn0, ws1, wn1, ws2, wn2)],
        out_specs=pl.BlockSpec((1, TB, 512), lambda i: (i, 0, 0)),
        out_shape=jax.ShapeDtypeStruct((nt, TB, 512), jnp.float32),
    )(self_vec, sums, *ws)


def kernel(batch, neigh_samples, embedding,
           W_self_0, W_neigh_0, W_self_1, W_neigh_1, W_self_2, W_neigh_2):
    NSEG = 2
    segsz = B // NSEG
    epw = segsz // NW
    inv = jnp.float32(1.0 / NEIGH)
    wn = (W_neigh_0 * inv, W_neigh_1 * inv, W_neigh_2 * inv)
    batch4d = batch.reshape(NSEG, NW, epw // 128, 128)
    neigh5d = neigh_samples.reshape(NL, NSEG, NW, epw // CH, ROWS)
    sc_outs = [_sc_gather_mean(batch4d, neigh5d, embedding, epw, s)
               for s in range(NSEG)]
    outs = [_tc_mlp(sv, sm, W_self_0, wn[0], W_self_1, wn[1],
                    W_self_2, wn[2]) for sv, sm in sc_outs]
    return jnp.concatenate(outs, axis=0).reshape(B, 512)


# speedup vs baseline: 1.1702x; 1.1702x over previous
import functools

import jax
import jax.numpy as jnp
from jax import lax
from jax.experimental import pallas as pl
from jax.experimental.pallas import tpu as pltpu
from jax.experimental.pallas import tpu_sc as plsc

B = 16384
E = 128
NEIGH = 25
NL = 3
NW = 32
EPW = B // NW
CH = 4
ROWS = CH * NEIGH
NCH = EPW // CH
TB = 2048
NT = B // TB
WPT = TB // EPW


def _sc_gather_mean(batch2d, neigh4d, embedding):
    mesh = plsc.VectorSubcoreMesh(core_axis_name="c", subcore_axis_name="s")
    n_self = EPW // 128

    @functools.partial(
        pl.kernel,
        out_type=(
            jax.ShapeDtypeStruct((NT, TB, E), jnp.float32),
            jax.ShapeDtypeStruct((NT, NL, TB, E), jnp.float32),
        ),
        mesh=mesh,
        scratch_types=[
            pltpu.VMEM((n_self, 128), jnp.int32),
            pltpu.VMEM((NCH // 2, ROWS), jnp.int32),
        ] + [pltpu.VMEM((ROWS, E), jnp.float32) for _ in range(4)] + [
            pltpu.VMEM((EPW, E), jnp.float32),
        ] + [pltpu.SemaphoreType.DMA for _ in range(4)],
    )
    def k(batch_hbm, neigh_hbm, emb_hbm, out_self, out_sums,
          sidx_v, idx_v, r0b, r1b, r2b, r3b, out_v,
          s0, s1, s2, s3):
        wid = lax.axis_index("s") * 2 + lax.axis_index("c")
        tile = wid // WPT
        trow = (wid % WPT) * EPW
        bufs = (r0b, r1b, r2b, r3b)
        sems = (s0, s1, s2, s3)

        pltpu.sync_copy(batch_hbm.at[wid], sidx_v)
        for c in range(n_self):
            pltpu.async_copy(emb_hbm.at[sidx_v.at[c]],
                             out_v.at[pl.ds(c * 128, 128), :], s0)
        for c in range(n_self):
            pltpu.make_async_copy(emb_hbm.at[sidx_v.at[c]],
                                  out_v.at[pl.ds(c * 128, 128), :], s0).wait()
        pltpu.sync_copy(out_v, out_self.at[tile, pl.ds(trow, EPW), :])

        def accumulate(buf, g):
            for e in range(CH):
                r0 = e * NEIGH
                accs = tuple(buf[r0, pl.ds(r * 16, 16)] for r in range(8))

                def jbody(j, a):
                    return tuple(a[r] + buf[j, pl.ds(r * 16, 16)]
                                 for r in range(8))

                accs = lax.fori_loop(r0 + 1, r0 + NEIGH, jbody, accs,
                                     unroll=4)
                orow = g * CH + e
                for r in range(8):
                    out_v[orow, pl.ds(r * 16, 16)] = accs[r]

        NB = 4
        HC = NCH // 2
        for layer in range(NL):
            for half in range(2):
                pltpu.sync_copy(neigh_hbm.at[layer, wid, pl.ds(half * HC, HC)],
                                idx_v)
                for b in range(NB):
                    pltpu.async_copy(emb_hbm.at[idx_v.at[b]], bufs[b], sems[b])

                def pbody(p, _):
                    for b in range(NB):
                        c = NB * p + b
                        pltpu.make_async_copy(emb_hbm.at[idx_v.at[c]],
                                              bufs[b], sems[b]).wait()
                        accumulate(bufs[b], half * HC + c)

                        @pl.when(c + NB < HC)
                        def _():
                            pltpu.async_copy(emb_hbm.at[idx_v.at[c + NB]],
                                             bufs[b], sems[b])
                    return 0

                lax.fori_loop(0, HC // NB, pbody, 0)
            pltpu.sync_copy(out_v,
                            out_sums.at[tile, layer, pl.ds(trow, EPW), :])

    return k(batch2d, neigh4d, embedding)


def _tc_mlp(self_vec, sums, ws0, wn0, ws1, wn1, ws2, wn2):
    def body(s_ref, m_ref, ws0r, wn0r, ws1r, wn1r, ws2r, wn2r, o_ref):
        def dot(a, w):
            return jnp.dot(a, w, preferred_element_type=jnp.float32)

        h = jnp.maximum(
            dot(s_ref[0], ws0r[0])
            + dot(m_ref[0, 0], wn0r[0]), 0.0)
        h = jnp.maximum(
            dot(h, ws1r[0])
            + dot(m_ref[0, 1], wn1r[0]), 0.0)
        o_ref[0] = jnp.maximum(
            dot(h, ws2r[0])
            + dot(m_ref[0, 2], wn2r[0]), 0.0)

    def wspec(w):
        return pl.BlockSpec((1,) + w.shape, lambda i: (0, 0, 0))

    ws = [w[None] for w in (ws0, wn0, ws1, wn1, ws2, wn2)]
    return pl.pallas_call(
        body,
        grid=(NT,),
        in_specs=[
            pl.BlockSpec((1, TB, E), lambda i: (i, 0, 0)),
            pl.BlockSpec((1, NL, TB, E), lambda i: (i, 0, 0, 0)),
        ] + [wspec(w) for w in (ws0, wn0, ws1, wn1, ws2, wn2)],
        out_specs=pl.BlockSpec((1, TB, 512), lambda i: (i, 0, 0)),
        out_shape=jax.ShapeDtypeStruct((NT, TB, 512), jnp.float32),
    )(self_vec, sums, *ws).reshape(B, 512)


def kernel(batch, neigh_samples, embedding,
           W_self_0, W_neigh_0, W_self_1, W_neigh_1, W_self_2, W_neigh_2):
    batch2d = batch.reshape(NW, EPW // 128, 128)
    neigh4d = neigh_samples.reshape(NL, NW, NCH, ROWS)
    self_vec, sums = _sc_gather_mean(batch2d, neigh4d, embedding)
    inv = jnp.float32(1.0 / NEIGH)
    return _tc_mlp(self_vec, sums,
                   W_self_0, W_neigh_0 * inv,
                   W_self_1, W_neigh_1 * inv,
                   W_self_2, W_neigh_2 * inv)
